# baseline (speedup 1.0000x reference)
import jax
import jax.numpy as jnp
from jax.experimental import pallas as pl
from jax.experimental.pallas import tpu as pltpu

N = 10000
IN_CH = 3
EMB_DIM = 8
HID = 8
KNN = 50
R = 100.0
N_ITERS = 4
CHUNK = 1000


def _ln(h, g, b):
    m = jnp.mean(h, axis=-1, keepdims=True)
    v = jnp.var(h, axis=-1, keepdims=True)
    return (h - m) / jnp.sqrt(v + 1e-5) * g + b


def _emb_apply(p, x):
    for (W, b) in p["layers"]:
        x = jnp.tanh(x @ W + b)
    W, b = p["emb"]
    return x @ W + b


def _edge_net(p, x, start, end):
    h = jnp.concatenate([x[start], x[end]], axis=1)
    for i in range(3):
        W, b = p["lin"][i]
        g, be = p["ln"][i]
        h = jnp.tanh(_ln(h @ W + b, g, be))
    W, b = p["lin"][3]
    return (h @ W + b)[:, 0]


def _node_net(p, x, e, start, end, mask):
    ew = e * mask
    mi = jax.ops.segment_sum(ew[:, None] * x[start], end, num_segments=x.shape[0])
    mo = jax.ops.segment_sum(ew[:, None] * x[end], start, num_segments=x.shape[0])
    h = jnp.concatenate([mi, mo, x], axis=1)
    for i in range(3):
        W, b = p["lin"][i]
        g, be = p["ln"][i]
        h = jnp.tanh(_ln(h @ W + b, g, be))
    W, b = p["lin"][3]
    return h @ W + b


_KNN_B = 400


def _knn_kernel(d_ref, dist_ref, idx_ref, s_ref):
    B = d_ref.shape[0]
    npad = d_ref.shape[1]
    s_ref[...] = d_ref[...]
    iota = jax.lax.broadcasted_iota(jnp.int32, (B, npad), 1)
    kio = jax.lax.broadcasted_iota(jnp.int32, (B, KNN), 1)
    big = jnp.int32(2**30)

    def body(it, carry):
        acc_d, acc_i = carry
        S = s_ref[...]
        v = jnp.min(S, axis=1, keepdims=True)
        cand = jnp.where(S == v, iota, big)
        i = jnp.min(cand, axis=1, keepdims=True)
        s_ref[...] = jnp.where(iota == i, jnp.inf, S)
        sel = (kio == it)
        acc_d = jnp.where(sel, v, acc_d)
        acc_i = jnp.where(sel, i, acc_i)
        return acc_d, acc_i

    acc_d = jnp.zeros((B, KNN), jnp.float32)
    acc_i = jnp.zeros((B, KNN), jnp.int32)
    acc_d, acc_i = jax.lax.fori_loop(0, KNN, body, (acc_d, acc_i))
    dist_ref[...] = acc_d
    idx_ref[...] = acc_i


def _knn_pallas(spatial):
    n = spatial.shape[0]
    sq = jnp.sum(spatial * spatial, axis=1)
    ds = []
    for s in range(0, n, CHUNK):
        q = spatial[s:s + CHUNK]
        ds.append(jnp.sum(q * q, axis=1)[:, None] - 2.0 * (q @ spatial.T) + sq[None, :])
    d = jnp.concatenate(ds, axis=0)
    dist, idx = pl.pallas_call(
        _knn_kernel,
        grid=(n // _KNN_B,),
        in_specs=[
            pl.BlockSpec((_KNN_B, n), lambda i: (i, 0)),
        ],
        out_specs=[
            pl.BlockSpec((_KNN_B, KNN), lambda i: (i, 0)),
            pl.BlockSpec((_KNN_B, KNN), lambda i: (i, 0)),
        ],
        out_shape=[
            jax.ShapeDtypeStruct((n, KNN), jnp.float32),
            jax.ShapeDtypeStruct((n, KNN), jnp.int32),
        ],
        scratch_shapes=[pltpu.VMEM((_KNN_B, n), jnp.float32)],
    )(d)
    return dist, idx


def _build_edges(spatial, layers):
    n = spatial.shape[0]
    dist, idx = _knn_pallas(spatial)
    end = jnp.repeat(jnp.arange(n), KNN)
    start = idx.reshape(-1)
    mask = (dist.reshape(-1) < R * R) & ((layers[end] - layers[start]) == 1)
    return start, end, mask.astype(jnp.float32)


def _edge_mlp_kernel(h_ref, m_ref, w0, b0, g0, be0, w1, b1, g1, be1,
                     w2, b2, g2, be2, w3, b3, o_ref):
    h = h_ref[...]
    for (W, b, g, be) in ((w0, b0, g0, be0), (w1, b1, g1, be1), (w2, b2, g2, be2)):
        h = jnp.dot(h, W[...], preferred_element_type=jnp.float32) + b[...][None, :]
        mu = jnp.mean(h, axis=-1, keepdims=True)
        v = jnp.var(h, axis=-1, keepdims=True)
        h = jnp.tanh((h - mu) / jnp.sqrt(v + 1e-5) * g[...][None, :] + be[...][None, :])
    h = jnp.dot(h, w3[...], preferred_element_type=jnp.float32) + b3[...][None, :]
    o_ref[...] = (h[:, 0] * m_ref[0, 0, :])[None, None, :]


def _edge_logits_pallas(p, h0, mask):
    E = h0.shape[0]
    BLK = 5000
    G = E // BLK
    flat = []
    for i in range(3):
        W, b = p["lin"][i]
        g, be = p["ln"][i]
        flat += [W, b, g, be]
    W3, b3 = p["lin"][3]
    flat += [W3, b3]
    specs = [pl.BlockSpec((BLK, 16), lambda i: (i, 0)),
             pl.BlockSpec((1, 1, BLK), lambda i: (i, 0, 0))]
    for a in flat:
        if a.ndim == 2:
            specs.append(pl.BlockSpec(a.shape, lambda i: (0, 0)))
        else:
            specs.append(pl.BlockSpec(a.shape, lambda i: (0,)))
    out = pl.pallas_call(
        _edge_mlp_kernel,
        grid=(G,),
        in_specs=specs,
        out_specs=pl.BlockSpec((1, 1, BLK), lambda i: (i, 0, 0)),
        out_shape=jax.ShapeDtypeStruct((G, 1, BLK), jnp.float32),
    )(h0, mask.reshape(G, 1, BLK), *flat)
    return out.reshape(E)


def kernel(x, layers, params):
    spatial = _emb_apply(params["emb1"], x)
    s1, e1, m1 = _build_edges(spatial, layers)
    W, b = params["ifn"]["lin"][0]
    g, be = params["ifn"]["ln"][0]
    f = jnp.tanh(_ln(jnp.concatenate([spatial, x], axis=-1) @ W + b, g, be))
    for _ in range(N_ITERS // 2):
        f0 = f
        e = jax.nn.sigmoid(_edge_net(params["edge"], f, s1, e1))
        f = _node_net(params["node"], f, e, s1, e1, m1) + f0
    spatial2 = _emb_apply(params["emb2"], jnp.concatenate([spatial, x, f], axis=-1))
    s2, e2, m2 = _build_edges(spatial2, layers)
    for _ in range(N_ITERS // 2):
        f0 = f
        e = jax.nn.sigmoid(_edge_net(params["edge"], f, s2, e2))
        f = _node_net(params["node"], f, e, s2, e2, m2) + f0
    h0 = jnp.concatenate([f[s2], f[e2]], axis=1)
    logits = _edge_logits_pallas(params["edge"], h0, m2)
    ratio = jnp.sum(m2) / spatial2.shape[0]
    return logits, spatial2, jnp.stack([s2, e2]), ratio
